# Initial kernel scaffold; baseline (speedup 1.0000x reference)
import jax
import jax.numpy as jnp
from jax.experimental import pallas as pl


def kernel(x, edge_index, W1, b1, Wg, bg, W2, b2):
    raise NotImplementedError("write your pallas kernel here")


# speedup vs baseline: 51.0069x; 51.0069x over previous
import functools

import jax
import jax.numpy as jnp
from jax import lax
from jax.experimental import pallas as pl
from jax.experimental.pallas import tpu as pltpu
from jax.experimental.pallas import tpu_sc as plsc

N_NODES = 100000
N_EDGES = 3200000
D_FEAT = 128
HIDDEN = 16
N_CLASSES = 16

NC, NS = 2, 16
LANE = 128
CHUNK_ROWS = 16
EDGES_PER_CHUNK = CHUNK_ROWS * LANE
E_PAD = 3211264
E_ROWS = E_PAD // LANE
ROWS_PER_T = E_ROWS // NS
CHUNKS_PER_T = ROWS_PER_T // CHUNK_ROWS

NODES_PER_CORE = N_NODES // NC
JUNK_MASK = 2047
ACC_ROWS = 52224
ROWS_PER_SUB = ACC_ROWS // NS
ZCHUNK = 1632

_mesh = lambda: plsc.VectorSubcoreMesh(
    core_axis_name="c", subcore_axis_name="s", num_cores=NC, num_subcores=NS)


def _remap_rows(idx_ref, base):
    def row_body(j, _):
        def sub_body(k, _):
            v = idx_ref[j, pl.ds(k * 16, 16)]
            local = v - base
            ok = (local >= 0) & (local < NODES_PER_CORE)
            junk = NODES_PER_CORE + jnp.bitwise_and(v, JUNK_MASK)
            idx_ref[j, pl.ds(k * 16, 16)] = jnp.where(ok, local, junk)
            return 0
        return lax.fori_loop(0, LANE // 16, sub_body, 0)
    lax.fori_loop(0, CHUNK_ROWS, row_body, 0)


@functools.partial(
    pl.kernel,
    out_type=jax.ShapeDtypeStruct((NC, ACC_ROWS), jnp.float32),
    mesh=_mesh(),
    scratch_types=[
        pltpu.VMEM((CHUNK_ROWS, LANE), jnp.int32),
        pltpu.VMEM((LANE,), jnp.float32),
        pltpu.VMEM((ROWS_PER_SUB,), jnp.float32),
        pltpu.VMEM_SHARED((ACC_ROWS,), jnp.float32),
    ],
    compiler_params=pltpu.CompilerParams(use_tc_tiling_on_sc=False),
)
def _deg_kernel(dst_hbm, out_hbm, idx_v, ones_v, zeros_v, acc):
    cid = lax.axis_index("c")
    sid = lax.axis_index("s")

    def fill_ones(i, _):
        ones_v[pl.ds(i * 16, 16)] = jnp.ones((16,), jnp.float32)
        return 0
    lax.fori_loop(0, LANE // 16, fill_ones, 0)

    def fill_zeros(i, _):
        zeros_v[pl.ds(i * 16, 16)] = jnp.zeros((16,), jnp.float32)
        return 0
    lax.fori_loop(0, ROWS_PER_SUB // 16, fill_zeros, 0)

    pltpu.sync_copy(zeros_v, acc.at[pl.ds(sid * ROWS_PER_SUB, ROWS_PER_SUB)])
    plsc.subcore_barrier()

    base = cid * NODES_PER_CORE
    row0 = sid * ROWS_PER_T

    def chunk_body(c, _):
        pltpu.sync_copy(dst_hbm.at[pl.ds(row0 + c * CHUNK_ROWS, CHUNK_ROWS)], idx_v)
        _remap_rows(idx_v, base)
        for j in range(CHUNK_ROWS):
            pltpu.sync_copy(ones_v, acc.at[idx_v.at[j]], add=True)
        return 0
    lax.fori_loop(0, CHUNKS_PER_T, chunk_body, 0)

    plsc.subcore_barrier()
    pltpu.sync_copy(acc.at[pl.ds(sid * ROWS_PER_SUB, ROWS_PER_SUB)],
                    out_hbm.at[cid].at[pl.ds(sid * ROWS_PER_SUB, ROWS_PER_SUB)])


@functools.partial(
    pl.kernel,
    out_type=jax.ShapeDtypeStruct((NC, ACC_ROWS, HIDDEN), jnp.float32),
    mesh=_mesh(),
    scratch_types=[
        pltpu.VMEM((CHUNK_ROWS, LANE), jnp.int32),
        pltpu.VMEM((CHUNK_ROWS, LANE), jnp.int32),
        pltpu.VMEM((CHUNK_ROWS, LANE, HIDDEN), jnp.float32),
        pltpu.VMEM((ZCHUNK, HIDDEN), jnp.float32),
        pltpu.VMEM_SHARED((ACC_ROWS, HIDDEN), jnp.float32),
        pltpu.SemaphoreType.DMA,
    ],
    compiler_params=pltpu.CompilerParams(use_tc_tiling_on_sc=False),
)
def _agg_kernel(y_hbm, src_hbm, dst_hbm, out_hbm, sidx, didx, rows, zeros_v, acc, sem):
    cid = lax.axis_index("c")
    sid = lax.axis_index("s")

    def fill_zeros(i, _):
        zeros_v[i] = jnp.zeros((HIDDEN,), jnp.float32)
        return 0
    lax.fori_loop(0, ZCHUNK, fill_zeros, 0)

    zbase = sid * ROWS_PER_SUB
    for k in range(ROWS_PER_SUB // ZCHUNK):
        pltpu.sync_copy(zeros_v, acc.at[pl.ds(zbase + k * ZCHUNK, ZCHUNK)])
    plsc.subcore_barrier()

    base = cid * NODES_PER_CORE
    row0 = sid * ROWS_PER_T

    def chunk_body(c, _):
        r = row0 + c * CHUNK_ROWS
        pltpu.sync_copy(src_hbm.at[pl.ds(r, CHUNK_ROWS)], sidx)
        pltpu.sync_copy(dst_hbm.at[pl.ds(r, CHUNK_ROWS)], didx)
        cps = [pltpu.async_copy(y_hbm.at[sidx.at[j]], rows.at[j], sem)
               for j in range(CHUNK_ROWS)]
        _remap_rows(didx, base)
        for cp in cps:
            cp.wait()
        for j in range(CHUNK_ROWS):
            pltpu.sync_copy(rows.at[j], acc.at[didx.at[j]], add=True)
        return 0
    lax.fori_loop(0, CHUNKS_PER_T, chunk_body, 0)

    plsc.subcore_barrier()
    ob = sid * ROWS_PER_SUB
    pltpu.sync_copy(acc.at[pl.ds(ob, ROWS_PER_SUB)],
                    out_hbm.at[cid].at[pl.ds(ob, ROWS_PER_SUB)])


BLK = 1000
GRID = N_NODES // BLK


def _y_body(x_ref, w1t_ref, b1_ref, wgt_ref, deg_ref, y_ref):
    h = jnp.dot(x_ref[...], w1t_ref[...], preferred_element_type=jnp.float32)
    h = jnp.maximum(h + b1_ref[...], 0.0)
    xw = jnp.dot(h, wgt_ref[...], preferred_element_type=jnp.float32)
    dinv = lax.rsqrt(deg_ref[...][:, 0] + 1.0)
    y_ref[...] = xw * dinv[:, None]


_y_call = pl.pallas_call(
    _y_body,
    grid=(GRID,),
    in_specs=[
        pl.BlockSpec((BLK, D_FEAT), lambda i: (i, 0)),
        pl.BlockSpec((D_FEAT, HIDDEN), lambda i: (0, 0)),
        pl.BlockSpec((1, HIDDEN), lambda i: (0, 0)),
        pl.BlockSpec((HIDDEN, HIDDEN), lambda i: (0, 0)),
        pl.BlockSpec((BLK, 1), lambda i: (i, 0)),
    ],
    out_specs=pl.BlockSpec((BLK, HIDDEN), lambda i: (i, 0)),
    out_shape=jax.ShapeDtypeStruct((N_NODES, HIDDEN), jnp.float32),
)


def _head_body(y_ref, agg_ref, deg_ref, bg_ref, w2t_ref, b2_ref, o_ref):
    dinv = lax.rsqrt(deg_ref[...][:, 0] + 1.0)
    s = (agg_ref[...] + y_ref[...]) * dinv[:, None] + bg_ref[...]
    r = jnp.maximum(s, 0.0)
    logits = jnp.dot(r, w2t_ref[...], preferred_element_type=jnp.float32) + b2_ref[...]
    m = jnp.max(logits, axis=1, keepdims=True)
    e = jnp.exp(logits - m)
    lse = jnp.log(jnp.sum(e, axis=1, keepdims=True)) + m
    o_ref[...] = logits - lse


_head_call = pl.pallas_call(
    _head_body,
    grid=(GRID,),
    in_specs=[
        pl.BlockSpec((BLK, HIDDEN), lambda i: (i, 0)),
        pl.BlockSpec((BLK, HIDDEN), lambda i: (i, 0)),
        pl.BlockSpec((BLK, 1), lambda i: (i, 0)),
        pl.BlockSpec((1, HIDDEN), lambda i: (0, 0)),
        pl.BlockSpec((HIDDEN, N_CLASSES), lambda i: (0, 0)),
        pl.BlockSpec((1, N_CLASSES), lambda i: (0, 0)),
    ],
    out_specs=pl.BlockSpec((BLK, N_CLASSES), lambda i: (i, 0)),
    out_shape=jax.ShapeDtypeStruct((N_NODES, N_CLASSES), jnp.float32),
)


def kernel(x, edge_index, W1, b1, Wg, bg, W2, b2):
    src = edge_index[0].astype(jnp.int32)
    dst = edge_index[1].astype(jnp.int32)
    npad = E_PAD - N_EDGES
    pad_i = jnp.arange(npad, dtype=jnp.int32)
    src_p = jnp.concatenate([src, pad_i % N_NODES]).reshape(E_ROWS, LANE)
    dst_p = jnp.concatenate([dst, N_NODES + (pad_i & JUNK_MASK)]).reshape(E_ROWS, LANE)

    deg_parts = _deg_kernel(dst_p)
    deg = jnp.concatenate(
        [deg_parts[0, :NODES_PER_CORE], deg_parts[1, :NODES_PER_CORE]]
    ).reshape(N_NODES, 1)
    y = _y_call(x, W1.T, b1.reshape(1, HIDDEN), Wg.T, deg)
    agg_parts = _agg_kernel(y, src_p, dst_p)
    agg = jnp.concatenate(
        [agg_parts[0, :NODES_PER_CORE], agg_parts[1, :NODES_PER_CORE]], axis=0)
    return _head_call(y, agg, deg, bg.reshape(1, HIDDEN),
                      W2.T, b2.reshape(1, N_CLASSES))


# speedup vs baseline: 66.8273x; 1.3102x over previous
import functools

import jax
import jax.numpy as jnp
from jax import lax
from jax.experimental import pallas as pl
from jax.experimental.pallas import tpu as pltpu
from jax.experimental.pallas import tpu_sc as plsc

N_NODES = 100000
N_EDGES = 3200000
D_FEAT = 128
HIDDEN = 16
N_CLASSES = 16

NC, NS = 2, 16
LANE = 128
CHUNK_ROWS = 16
EDGES_PER_CHUNK = CHUNK_ROWS * LANE
E_PAD = 3211264
E_ROWS = E_PAD // LANE
ROWS_PER_T = E_ROWS // NS
CHUNKS_PER_T = ROWS_PER_T // CHUNK_ROWS

NODES_PER_CORE = N_NODES // NC
JUNK_MASK = 2047
ACC_ROWS = 52224
ROWS_PER_SUB = ACC_ROWS // NS
ZCHUNK = 1632

_mesh = lambda: plsc.VectorSubcoreMesh(
    core_axis_name="c", subcore_axis_name="s", num_cores=NC, num_subcores=NS)


def _remap_rows(idx_ref, base):
    def row_body(j, _):
        def sub_body(k, _):
            v = idx_ref[j, pl.ds(k * 16, 16)]
            local = v - base
            ok = (local >= 0) & (local < NODES_PER_CORE)
            junk = NODES_PER_CORE + jnp.bitwise_and(v, JUNK_MASK)
            idx_ref[j, pl.ds(k * 16, 16)] = jnp.where(ok, local, junk)
            return 0
        return lax.fori_loop(0, LANE // 16, sub_body, 0)
    lax.fori_loop(0, CHUNK_ROWS, row_body, 0)


@functools.partial(
    pl.kernel,
    out_type=jax.ShapeDtypeStruct((NC, ACC_ROWS), jnp.float32),
    mesh=_mesh(),
    scratch_types=[
        pltpu.VMEM((CHUNK_ROWS, LANE), jnp.int32),
        pltpu.VMEM((LANE,), jnp.float32),
        pltpu.VMEM((ROWS_PER_SUB,), jnp.float32),
        pltpu.VMEM_SHARED((ACC_ROWS,), jnp.float32),
        pltpu.SemaphoreType.DMA,
    ],
    compiler_params=pltpu.CompilerParams(use_tc_tiling_on_sc=False),
)
def _deg_kernel(dst_hbm, out_hbm, idx_v, ones_v, zeros_v, acc, sem):
    cid = lax.axis_index("c")
    sid = lax.axis_index("s")

    def fill_ones(i, _):
        ones_v[pl.ds(i * 16, 16)] = jnp.ones((16,), jnp.float32)
        return 0
    lax.fori_loop(0, LANE // 16, fill_ones, 0)

    def fill_zeros(i, _):
        zeros_v[pl.ds(i * 16, 16)] = jnp.zeros((16,), jnp.float32)
        return 0
    lax.fori_loop(0, ROWS_PER_SUB // 16, fill_zeros, 0)

    pltpu.sync_copy(zeros_v, acc.at[pl.ds(sid * ROWS_PER_SUB, ROWS_PER_SUB)])
    plsc.subcore_barrier()

    base = cid * NODES_PER_CORE
    row0 = sid * ROWS_PER_T

    def chunk_body(c, _):
        pltpu.sync_copy(dst_hbm.at[pl.ds(row0 + c * CHUNK_ROWS, CHUNK_ROWS)], idx_v)
        _remap_rows(idx_v, base)
        cps = [pltpu.async_copy(ones_v, acc.at[idx_v.at[j]], add=True, sem=sem)
               for j in range(CHUNK_ROWS)]
        for cp in cps:
            cp.wait()
        return 0
    lax.fori_loop(0, CHUNKS_PER_T, chunk_body, 0)

    plsc.subcore_barrier()
    pltpu.sync_copy(acc.at[pl.ds(sid * ROWS_PER_SUB, ROWS_PER_SUB)],
                    out_hbm.at[cid].at[pl.ds(sid * ROWS_PER_SUB, ROWS_PER_SUB)])


@functools.partial(
    pl.kernel,
    out_type=jax.ShapeDtypeStruct((NC, ACC_ROWS, HIDDEN), jnp.float32),
    mesh=_mesh(),
    scratch_types=[
        pltpu.VMEM((CHUNK_ROWS, LANE), jnp.int32),
        pltpu.VMEM((CHUNK_ROWS, LANE), jnp.int32),
        pltpu.VMEM((CHUNK_ROWS, LANE, HIDDEN), jnp.float32),
        pltpu.VMEM((ZCHUNK, HIDDEN), jnp.float32),
        pltpu.VMEM_SHARED((ACC_ROWS, HIDDEN), jnp.float32),
        pltpu.SemaphoreType.DMA,
        pltpu.SemaphoreType.DMA,
        pltpu.SemaphoreType.DMA,
        pltpu.SemaphoreType.DMA,
    ],
    compiler_params=pltpu.CompilerParams(use_tc_tiling_on_sc=False),
)
def _agg_kernel(y_hbm, src_hbm, dst_hbm, out_hbm, sidx, didx, rows, zeros_v, acc,
                sem0, sem1, sem2, sem3):
    cid = lax.axis_index("c")
    sid = lax.axis_index("s")

    def fill_zeros(i, _):
        zeros_v[i] = jnp.zeros((HIDDEN,), jnp.float32)
        return 0
    lax.fori_loop(0, ZCHUNK, fill_zeros, 0)

    zbase = sid * ROWS_PER_SUB
    for k in range(ROWS_PER_SUB // ZCHUNK):
        pltpu.sync_copy(zeros_v, acc.at[pl.ds(zbase + k * ZCHUNK, ZCHUNK)])
    plsc.subcore_barrier()

    base = cid * NODES_PER_CORE
    row0 = sid * ROWS_PER_T

    sems = [sem0, sem1, sem2, sem3]
    GR = CHUNK_ROWS // 4

    def chunk_body(c, _):
        r = row0 + c * CHUNK_ROWS
        pltpu.sync_copy(src_hbm.at[pl.ds(r, CHUNK_ROWS)], sidx)
        pltpu.sync_copy(dst_hbm.at[pl.ds(r, CHUNK_ROWS)], didx)
        cps = [pltpu.async_copy(y_hbm.at[sidx.at[j]], rows.at[j], sems[j // GR])
               for j in range(CHUNK_ROWS)]
        _remap_rows(didx, base)
        for g in range(4):
            for j in range(g * GR, (g + 1) * GR):
                cps[j].wait()
            for j in range(g * GR, (g + 1) * GR):
                pltpu.sync_copy(rows.at[j], acc.at[didx.at[j]], add=True)
        return 0
    lax.fori_loop(0, CHUNKS_PER_T, chunk_body, 0)

    plsc.subcore_barrier()
    ob = sid * ROWS_PER_SUB
    pltpu.sync_copy(acc.at[pl.ds(ob, ROWS_PER_SUB)],
                    out_hbm.at[cid].at[pl.ds(ob, ROWS_PER_SUB)])


BLK = 2000
GRID = N_NODES // BLK
BLKS_PER_CORE = NODES_PER_CORE // BLK


def _y_body(x_ref, w1t_ref, b1_ref, wgt_ref, deg_ref, y_ref):
    h = jnp.dot(x_ref[...], w1t_ref[...], preferred_element_type=jnp.float32)
    h = jnp.maximum(h + b1_ref[...], 0.0)
    xw = jnp.dot(h, wgt_ref[...], preferred_element_type=jnp.float32)
    dinv = lax.rsqrt(deg_ref[...][:, 0] + 1.0)
    y_ref[...] = xw * dinv[:, None]


_y_call = pl.pallas_call(
    _y_body,
    grid=(GRID,),
    in_specs=[
        pl.BlockSpec((BLK, D_FEAT), lambda i: (i, 0)),
        pl.BlockSpec((D_FEAT, HIDDEN), lambda i: (0, 0)),
        pl.BlockSpec((1, HIDDEN), lambda i: (0, 0)),
        pl.BlockSpec((HIDDEN, HIDDEN), lambda i: (0, 0)),
        pl.BlockSpec((BLK, 1), lambda i: (i, 0)),
    ],
    out_specs=pl.BlockSpec((BLK, HIDDEN), lambda i: (i, 0)),
    out_shape=jax.ShapeDtypeStruct((N_NODES, HIDDEN), jnp.float32),
)


def _head_body(y_ref, agg_ref, deg_ref, bg_ref, w2t_ref, b2_ref, o_ref):
    dinv = lax.rsqrt(deg_ref[...][:, 0] + 1.0)
    s = (agg_ref[0] + y_ref[...]) * dinv[:, None] + bg_ref[...]
    r = jnp.maximum(s, 0.0)
    logits = jnp.dot(r, w2t_ref[...], preferred_element_type=jnp.float32) + b2_ref[...]
    m = jnp.max(logits, axis=1, keepdims=True)
    e = jnp.exp(logits - m)
    lse = jnp.log(jnp.sum(e, axis=1, keepdims=True)) + m
    o_ref[...] = logits - lse


_head_call = pl.pallas_call(
    _head_body,
    grid=(GRID,),
    in_specs=[
        pl.BlockSpec((BLK, HIDDEN), lambda i: (i, 0)),
        pl.BlockSpec((1, BLK, HIDDEN),
                     lambda i: (i // BLKS_PER_CORE, i % BLKS_PER_CORE, 0)),
        pl.BlockSpec((BLK, 1), lambda i: (i, 0)),
        pl.BlockSpec((1, HIDDEN), lambda i: (0, 0)),
        pl.BlockSpec((HIDDEN, N_CLASSES), lambda i: (0, 0)),
        pl.BlockSpec((1, N_CLASSES), lambda i: (0, 0)),
    ],
    out_specs=pl.BlockSpec((BLK, N_CLASSES), lambda i: (i, 0)),
    out_shape=jax.ShapeDtypeStruct((N_NODES, N_CLASSES), jnp.float32),
)


def kernel(x, edge_index, W1, b1, Wg, bg, W2, b2):
    src = edge_index[0].astype(jnp.int32)
    dst = edge_index[1].astype(jnp.int32)
    npad = E_PAD - N_EDGES
    pad_i = jnp.arange(npad, dtype=jnp.int32)
    src_p = jnp.concatenate([src, pad_i % N_NODES]).reshape(E_ROWS, LANE)
    dst_p = jnp.concatenate([dst, N_NODES + (pad_i & JUNK_MASK)]).reshape(E_ROWS, LANE)

    deg_parts = _deg_kernel(dst_p)
    deg = jnp.concatenate(
        [deg_parts[0, :NODES_PER_CORE], deg_parts[1, :NODES_PER_CORE]]
    ).reshape(N_NODES, 1)
    y = _y_call(x, W1.T, b1.reshape(1, HIDDEN), Wg.T, deg)
    agg_parts = _agg_kernel(y, src_p, dst_p)
    return _head_call(y, agg_parts, deg, bg.reshape(1, HIDDEN),
                      W2.T, b2.reshape(1, N_CLASSES))


# speedup vs baseline: 78.9772x; 1.1818x over previous
import functools

import jax
import jax.numpy as jnp
from jax import lax
from jax.experimental import pallas as pl
from jax.experimental.pallas import tpu as pltpu
from jax.experimental.pallas import tpu_sc as plsc

N_NODES = 100000
N_EDGES = 3200000
D_FEAT = 128
HIDDEN = 16
N_CLASSES = 16

NC, NS = 2, 16
LANE = 128
CHUNK_ROWS = 16
EDGES_PER_CHUNK = CHUNK_ROWS * LANE
E_PAD = 3211264
E_ROWS = E_PAD // LANE
ROWS_PER_T = E_ROWS // NS
CHUNKS_PER_T = ROWS_PER_T // CHUNK_ROWS

NODES_PER_CORE = N_NODES // NC
JUNK_MASK = 2047
ACC_ROWS = 52224
ROWS_PER_SUB = ACC_ROWS // NS
ZCHUNK = 1632

_mesh = lambda: plsc.VectorSubcoreMesh(
    core_axis_name="c", subcore_axis_name="s", num_cores=NC, num_subcores=NS)


def _remap_rows(idx_ref, base):
    def row_body(j, _):
        def sub_body(k, _):
            v = idx_ref[j, pl.ds(k * 16, 16)]
            local = v - base
            ok = (local >= 0) & (local < NODES_PER_CORE)
            junk = NODES_PER_CORE + jnp.bitwise_and(v, JUNK_MASK)
            idx_ref[j, pl.ds(k * 16, 16)] = jnp.where(ok, local, junk)
            return 0
        return lax.fori_loop(0, LANE // 16, sub_body, 0)
    lax.fori_loop(0, CHUNK_ROWS, row_body, 0)


@functools.partial(
    pl.kernel,
    out_type=jax.ShapeDtypeStruct((NC, ACC_ROWS), jnp.float32),
    mesh=_mesh(),
    scratch_types=[
        pltpu.VMEM((CHUNK_ROWS, LANE), jnp.int32),
        pltpu.VMEM((LANE,), jnp.float32),
        pltpu.VMEM((ROWS_PER_SUB,), jnp.float32),
        pltpu.VMEM_SHARED((ACC_ROWS,), jnp.float32),
        pltpu.SemaphoreType.DMA,
    ],
    compiler_params=pltpu.CompilerParams(use_tc_tiling_on_sc=False),
)
def _deg_kernel(dst_hbm, out_hbm, idx_v, ones_v, zeros_v, acc, sem):
    cid = lax.axis_index("c")
    sid = lax.axis_index("s")

    def fill_ones(i, _):
        ones_v[pl.ds(i * 16, 16)] = jnp.ones((16,), jnp.float32)
        return 0
    lax.fori_loop(0, LANE // 16, fill_ones, 0)

    def fill_zeros(i, _):
        zeros_v[pl.ds(i * 16, 16)] = jnp.zeros((16,), jnp.float32)
        return 0
    lax.fori_loop(0, ROWS_PER_SUB // 16, fill_zeros, 0)

    pltpu.sync_copy(zeros_v, acc.at[pl.ds(sid * ROWS_PER_SUB, ROWS_PER_SUB)])
    plsc.subcore_barrier()

    base = cid * NODES_PER_CORE
    row0 = sid * ROWS_PER_T

    def chunk_body(c, _):
        pltpu.sync_copy(dst_hbm.at[pl.ds(row0 + c * CHUNK_ROWS, CHUNK_ROWS)], idx_v)
        _remap_rows(idx_v, base)
        cps = [pltpu.async_copy(ones_v, acc.at[idx_v.at[j]], add=True, sem=sem)
               for j in range(CHUNK_ROWS)]
        for cp in cps:
            cp.wait()
        return 0
    lax.fori_loop(0, CHUNKS_PER_T, chunk_body, 0)

    plsc.subcore_barrier()
    pltpu.sync_copy(acc.at[pl.ds(sid * ROWS_PER_SUB, ROWS_PER_SUB)],
                    out_hbm.at[cid].at[pl.ds(sid * ROWS_PER_SUB, ROWS_PER_SUB)])


@functools.partial(
    pl.kernel,
    out_type=jax.ShapeDtypeStruct((NC, ACC_ROWS, HIDDEN), jnp.float32),
    mesh=_mesh(),
    scratch_types=[
        pltpu.VMEM((CHUNK_ROWS, LANE), jnp.int32),
        pltpu.VMEM((CHUNK_ROWS, LANE), jnp.int32),
        pltpu.VMEM((CHUNK_ROWS, LANE), jnp.int32),
        pltpu.VMEM((CHUNK_ROWS, LANE), jnp.int32),
        pltpu.VMEM((CHUNK_ROWS, LANE, HIDDEN), jnp.float32),
        pltpu.VMEM((CHUNK_ROWS, LANE, HIDDEN), jnp.float32),
        pltpu.VMEM_SHARED((ACC_ROWS, HIDDEN), jnp.float32),
        pltpu.SemaphoreType.DMA,
        pltpu.SemaphoreType.DMA,
    ],
    compiler_params=pltpu.CompilerParams(use_tc_tiling_on_sc=False),
)
def _agg_kernel(y_hbm, src_hbm, dst_hbm, out_hbm, sidx0, didx0, sidx1, didx1,
                rows0, rows1, acc, sem0, sem1):
    cid = lax.axis_index("c")
    sid = lax.axis_index("s")

    def zrow(j, _):
        def zlane(l, _):
            rows0[j, l] = jnp.zeros((HIDDEN,), jnp.float32)
            return 0
        return lax.fori_loop(0, LANE, zlane, 0)
    lax.fori_loop(0, CHUNK_ROWS, zrow, 0)

    zbase = sid * ROWS_PER_SUB
    for k in range(ROWS_PER_SUB // LANE):
        pltpu.sync_copy(rows0.at[0], acc.at[pl.ds(zbase + k * LANE, LANE)])
    rem = ROWS_PER_SUB % LANE
    pltpu.sync_copy(rows0.at[0].at[pl.ds(0, rem)],
                    acc.at[pl.ds(zbase + ROWS_PER_SUB - rem, rem)])
    plsc.subcore_barrier()

    base = cid * NODES_PER_CORE
    row0 = sid * ROWS_PER_T

    bufs = ((sidx0, didx0, rows0, sem0), (sidx1, didx1, rows1, sem1))

    def fire(c, buf):
        si, di, rw, sm = buf
        r = row0 + c * CHUNK_ROWS
        pltpu.sync_copy(src_hbm.at[pl.ds(r, CHUNK_ROWS)], si)
        pltpu.sync_copy(dst_hbm.at[pl.ds(r, CHUNK_ROWS)], di)
        for j in range(CHUNK_ROWS):
            pltpu.async_copy(y_hbm.at[si.at[j]], rw.at[j], sm)
        _remap_rows(di, base)

    def finish(buf):
        si, di, rw, sm = buf
        for j in range(CHUNK_ROWS):
            pltpu.make_async_copy(y_hbm.at[si.at[j]], rw.at[j], sm).wait()
        for j in range(CHUNK_ROWS):
            pltpu.sync_copy(rw.at[j], acc.at[di.at[j]], add=True)

    fire(0, bufs[0])

    def pair_body(i, _):
        fire(2 * i + 1, bufs[1])
        finish(bufs[0])
        fire(2 * i + 2, bufs[0])
        finish(bufs[1])
        return 0
    lax.fori_loop(0, CHUNKS_PER_T // 2 - 1, pair_body, 0)

    fire(CHUNKS_PER_T - 1, bufs[1])
    finish(bufs[0])
    finish(bufs[1])

    plsc.subcore_barrier()
    ob = sid * ROWS_PER_SUB
    pltpu.sync_copy(acc.at[pl.ds(ob, ROWS_PER_SUB)],
                    out_hbm.at[cid].at[pl.ds(ob, ROWS_PER_SUB)])


BLK = 2000
GRID = N_NODES // BLK
BLKS_PER_CORE = NODES_PER_CORE // BLK


def _xw_body(x_ref, w1t_ref, b1_ref, wgt_ref, xw_ref):
    h = jnp.dot(x_ref[...], w1t_ref[...], preferred_element_type=jnp.float32)
    h = jnp.maximum(h + b1_ref[...], 0.0)
    xw_ref[...] = jnp.dot(h, wgt_ref[...], preferred_element_type=jnp.float32)


_xw_call = pl.pallas_call(
    _xw_body,
    grid=(GRID,),
    in_specs=[
        pl.BlockSpec((BLK, D_FEAT), lambda i: (i, 0)),
        pl.BlockSpec((D_FEAT, HIDDEN), lambda i: (0, 0)),
        pl.BlockSpec((1, HIDDEN), lambda i: (0, 0)),
        pl.BlockSpec((HIDDEN, HIDDEN), lambda i: (0, 0)),
    ],
    out_specs=pl.BlockSpec((BLK, HIDDEN), lambda i: (i, 0)),
    out_shape=jax.ShapeDtypeStruct((N_NODES, HIDDEN), jnp.float32),
)


def _scale_body(xw_ref, deg_ref, y_ref):
    dinv = lax.rsqrt(deg_ref[...][:, 0] + 1.0)
    y_ref[...] = xw_ref[...] * dinv[:, None]


_scale_call = pl.pallas_call(
    _scale_body,
    grid=(GRID,),
    in_specs=[
        pl.BlockSpec((BLK, HIDDEN), lambda i: (i, 0)),
        pl.BlockSpec((BLK, 1), lambda i: (i, 0)),
    ],
    out_specs=pl.BlockSpec((BLK, HIDDEN), lambda i: (i, 0)),
    out_shape=jax.ShapeDtypeStruct((N_NODES, HIDDEN), jnp.float32),
)


def _head_body(y_ref, agg_ref, deg_ref, bg_ref, w2t_ref, b2_ref, o_ref):
    dinv = lax.rsqrt(deg_ref[...][:, 0] + 1.0)
    s = (agg_ref[0] + y_ref[...]) * dinv[:, None] + bg_ref[...]
    r = jnp.maximum(s, 0.0)
    logits = jnp.dot(r, w2t_ref[...], preferred_element_type=jnp.float32) + b2_ref[...]
    m = jnp.max(logits, axis=1, keepdims=True)
    e = jnp.exp(logits - m)
    lse = jnp.log(jnp.sum(e, axis=1, keepdims=True)) + m
    o_ref[...] = logits - lse


_head_call = pl.pallas_call(
    _head_body,
    grid=(GRID,),
    in_specs=[
        pl.BlockSpec((BLK, HIDDEN), lambda i: (i, 0)),
        pl.BlockSpec((1, BLK, HIDDEN),
                     lambda i: (i // BLKS_PER_CORE, i % BLKS_PER_CORE, 0)),
        pl.BlockSpec((BLK, 1), lambda i: (i, 0)),
        pl.BlockSpec((1, HIDDEN), lambda i: (0, 0)),
        pl.BlockSpec((HIDDEN, N_CLASSES), lambda i: (0, 0)),
        pl.BlockSpec((1, N_CLASSES), lambda i: (0, 0)),
    ],
    out_specs=pl.BlockSpec((BLK, N_CLASSES), lambda i: (i, 0)),
    out_shape=jax.ShapeDtypeStruct((N_NODES, N_CLASSES), jnp.float32),
)


def kernel(x, edge_index, W1, b1, Wg, bg, W2, b2):
    src = edge_index[0].astype(jnp.int32)
    dst = edge_index[1].astype(jnp.int32)
    npad = E_PAD - N_EDGES
    pad_i = jnp.arange(npad, dtype=jnp.int32)
    src_p = jnp.concatenate([src, pad_i % N_NODES]).reshape(E_ROWS, LANE)
    dst_p = jnp.concatenate([dst, N_NODES + (pad_i & JUNK_MASK)]).reshape(E_ROWS, LANE)

    deg_parts = _deg_kernel(dst_p)
    xw = _xw_call(x, W1.T, b1.reshape(1, HIDDEN), Wg.T)
    deg = jnp.concatenate(
        [deg_parts[0, :NODES_PER_CORE], deg_parts[1, :NODES_PER_CORE]]
    ).reshape(N_NODES, 1)
    y = _scale_call(xw, deg)
    agg_parts = _agg_kernel(y, src_p, dst_p)
    return _head_call(y, agg_parts, deg, bg.reshape(1, HIDDEN),
                      W2.T, b2.reshape(1, N_CLASSES))


# speedup vs baseline: 79.0420x; 1.0008x over previous
import functools

import jax
import jax.numpy as jnp
from jax import lax
from jax.experimental import pallas as pl
from jax.experimental.pallas import tpu as pltpu
from jax.experimental.pallas import tpu_sc as plsc

N_NODES = 100000
N_EDGES = 3200000
D_FEAT = 128
HIDDEN = 16
N_CLASSES = 16

NC, NS = 2, 16
LANE = 128
CHUNK_ROWS = 16
EDGES_PER_CHUNK = CHUNK_ROWS * LANE
E_PAD = 3211264
E_ROWS = E_PAD // LANE
ROWS_PER_T = E_ROWS // NS
CHUNKS_PER_T = ROWS_PER_T // CHUNK_ROWS

NODES_PER_CORE = N_NODES // NC
JUNK_MASK = 2047
ACC_ROWS = 52224
ROWS_PER_SUB = ACC_ROWS // NS

_mesh = lambda: plsc.VectorSubcoreMesh(
    core_axis_name="c", subcore_axis_name="s", num_cores=NC, num_subcores=NS)


def _remap_rows(idx_ref, base):
    def row_body(j, _):
        def sub_body(k, _):
            v = idx_ref[j, pl.ds(k * 16, 16)]
            local = v - base
            ok = (local >= 0) & (local < NODES_PER_CORE)
            junk = NODES_PER_CORE + jnp.bitwise_and(v, JUNK_MASK)
            idx_ref[j, pl.ds(k * 16, 16)] = jnp.where(ok, local, junk)
            return 0
        return lax.fori_loop(0, LANE // 16, sub_body, 0)
    lax.fori_loop(0, CHUNK_ROWS, row_body, 0)


@functools.partial(
    pl.kernel,
    out_type=jax.ShapeDtypeStruct((NC, ACC_ROWS), jnp.float32),
    mesh=_mesh(),
    scratch_types=[
        pltpu.VMEM((CHUNK_ROWS, LANE), jnp.int32),
        pltpu.VMEM((LANE,), jnp.float32),
        pltpu.VMEM((ROWS_PER_SUB,), jnp.float32),
        pltpu.VMEM_SHARED((ACC_ROWS,), jnp.float32),
        pltpu.SemaphoreType.DMA,
    ],
    compiler_params=pltpu.CompilerParams(use_tc_tiling_on_sc=False),
)
def _deg_kernel(dst_hbm, out_hbm, idx_v, ones_v, zeros_v, acc, sem):
    cid = lax.axis_index("c")
    sid = lax.axis_index("s")

    def fill_ones(i, _):
        ones_v[pl.ds(i * 16, 16)] = jnp.ones((16,), jnp.float32)
        return 0
    lax.fori_loop(0, LANE // 16, fill_ones, 0)

    def fill_zeros(i, _):
        zeros_v[pl.ds(i * 16, 16)] = jnp.zeros((16,), jnp.float32)
        return 0
    lax.fori_loop(0, ROWS_PER_SUB // 16, fill_zeros, 0)

    pltpu.sync_copy(zeros_v, acc.at[pl.ds(sid * ROWS_PER_SUB, ROWS_PER_SUB)])
    plsc.subcore_barrier()

    base = cid * NODES_PER_CORE
    row0 = sid * ROWS_PER_T

    def chunk_body(c, _):
        pltpu.sync_copy(dst_hbm.at[pl.ds(row0 + c * CHUNK_ROWS, CHUNK_ROWS)], idx_v)
        _remap_rows(idx_v, base)
        cps = [pltpu.async_copy(ones_v, acc.at[idx_v.at[j]], add=True, sem=sem)
               for j in range(CHUNK_ROWS)]
        for cp in cps:
            cp.wait()
        return 0
    lax.fori_loop(0, CHUNKS_PER_T, chunk_body, 0)

    plsc.subcore_barrier()
    pltpu.sync_copy(acc.at[pl.ds(sid * ROWS_PER_SUB, ROWS_PER_SUB)],
                    out_hbm.at[cid].at[pl.ds(sid * ROWS_PER_SUB, ROWS_PER_SUB)])


@functools.partial(
    pl.kernel,
    out_type=jax.ShapeDtypeStruct((NC, ACC_ROWS, HIDDEN), jnp.float32),
    mesh=_mesh(),
    scratch_types=[
        pltpu.VMEM((CHUNK_ROWS, LANE), jnp.int32),
        pltpu.VMEM((CHUNK_ROWS, LANE), jnp.int32),
        pltpu.VMEM((CHUNK_ROWS, LANE), jnp.int32),
        pltpu.VMEM((CHUNK_ROWS, LANE), jnp.int32),
        pltpu.VMEM((CHUNK_ROWS, LANE, HIDDEN), jnp.float32),
        pltpu.VMEM((CHUNK_ROWS, LANE, HIDDEN), jnp.float32),
        pltpu.VMEM_SHARED((ACC_ROWS, HIDDEN), jnp.float32),
        pltpu.SemaphoreType.DMA,
        pltpu.SemaphoreType.DMA,
    ],
    compiler_params=pltpu.CompilerParams(use_tc_tiling_on_sc=False),
)
def _agg_kernel(y_hbm, src_hbm, dst_hbm, out_hbm, sidx0, didx0, sidx1, didx1,
                rows0, rows1, acc, sem0, sem1):
    cid = lax.axis_index("c")
    sid = lax.axis_index("s")

    def zrow(j, _):
        def zlane(l, _):
            rows0[j, l] = jnp.zeros((HIDDEN,), jnp.float32)
            return 0
        return lax.fori_loop(0, LANE, zlane, 0)
    lax.fori_loop(0, CHUNK_ROWS, zrow, 0)

    zbase = sid * ROWS_PER_SUB
    for k in range(ROWS_PER_SUB // LANE):
        pltpu.sync_copy(rows0.at[0], acc.at[pl.ds(zbase + k * LANE, LANE)])
    rem = ROWS_PER_SUB % LANE
    pltpu.sync_copy(rows0.at[0].at[pl.ds(0, rem)],
                    acc.at[pl.ds(zbase + ROWS_PER_SUB - rem, rem)])
    plsc.subcore_barrier()

    base = cid * NODES_PER_CORE
    row0 = sid * ROWS_PER_T

    bufs = ((sidx0, didx0, rows0, sem0), (sidx1, didx1, rows1, sem1))

    def fire(c, buf):
        si, di, rw, sm = buf
        r = row0 + c * CHUNK_ROWS
        pltpu.sync_copy(src_hbm.at[pl.ds(r, CHUNK_ROWS)], si)
        pltpu.sync_copy(dst_hbm.at[pl.ds(r, CHUNK_ROWS)], di)
        for j in range(CHUNK_ROWS):
            pltpu.async_copy(y_hbm.at[si.at[j]], rw.at[j], sm)
        _remap_rows(di, base)

    def finish(buf):
        si, di, rw, sm = buf
        for j in range(CHUNK_ROWS):
            pltpu.make_async_copy(y_hbm.at[si.at[j]], rw.at[j], sm).wait()
        for j in range(CHUNK_ROWS):
            pltpu.sync_copy(rw.at[j], acc.at[di.at[j]], add=True)

    fire(0, bufs[0])

    def pair_body(i, _):
        fire(2 * i + 1, bufs[1])
        finish(bufs[0])
        fire(2 * i + 2, bufs[0])
        finish(bufs[1])
        return 0
    lax.fori_loop(0, CHUNKS_PER_T // 2 - 1, pair_body, 0)

    fire(CHUNKS_PER_T - 1, bufs[1])
    finish(bufs[0])
    finish(bufs[1])

    plsc.subcore_barrier()
    ob = sid * ROWS_PER_SUB
    pltpu.sync_copy(acc.at[pl.ds(ob, ROWS_PER_SUB)],
                    out_hbm.at[cid].at[pl.ds(ob, ROWS_PER_SUB)])


BLK = 2000
GRID = N_NODES // BLK
BLKS_PER_CORE = NODES_PER_CORE // BLK

_agg_map = lambda i: (i // BLKS_PER_CORE, i % BLKS_PER_CORE, 0)


def _xw_body(x_ref, w1t_ref, b1_ref, wgt_ref, xw_ref):
    h = jnp.dot(x_ref[...], w1t_ref[...], preferred_element_type=jnp.float32)
    h = jnp.maximum(h + b1_ref[...], 0.0)
    xw_ref[...] = jnp.dot(h, wgt_ref[...], preferred_element_type=jnp.float32)


_xw_call = pl.pallas_call(
    _xw_body,
    grid=(GRID,),
    in_specs=[
        pl.BlockSpec((BLK, D_FEAT), lambda i: (i, 0)),
        pl.BlockSpec((D_FEAT, HIDDEN), lambda i: (0, 0)),
        pl.BlockSpec((1, HIDDEN), lambda i: (0, 0)),
        pl.BlockSpec((HIDDEN, HIDDEN), lambda i: (0, 0)),
    ],
    out_specs=pl.BlockSpec((BLK, HIDDEN), lambda i: (i, 0)),
    out_shape=jax.ShapeDtypeStruct((N_NODES, HIDDEN), jnp.float32),
)


def _scale_body(xw_ref, deg_ref, y_ref):
    dinv = lax.rsqrt(deg_ref[...][:, 0] + 1.0)
    y_ref[...] = xw_ref[...] * dinv[:, None]


_scale_call = pl.pallas_call(
    _scale_body,
    grid=(GRID,),
    in_specs=[
        pl.BlockSpec((BLK, HIDDEN), lambda i: (i, 0)),
        pl.BlockSpec((BLK, 1), lambda i: (i, 0)),
    ],
    out_specs=pl.BlockSpec((BLK, HIDDEN), lambda i: (i, 0)),
    out_shape=jax.ShapeDtypeStruct((N_NODES, HIDDEN), jnp.float32),
)


def _head_body(y_ref, agg_ref, deg_ref, bg_ref, w2t_ref, b2_ref, o_ref):
    dinv = lax.rsqrt(deg_ref[...][:, 0] + 1.0)
    s = (agg_ref[0] + y_ref[...]) * dinv[:, None] + bg_ref[...]
    r = jnp.maximum(s, 0.0)
    logits = jnp.dot(r, w2t_ref[...], preferred_element_type=jnp.float32) + b2_ref[...]
    m = jnp.max(logits, axis=1, keepdims=True)
    e = jnp.exp(logits - m)
    lse = jnp.log(jnp.sum(e, axis=1, keepdims=True)) + m
    o_ref[...] = logits - lse


_head_call = pl.pallas_call(
    _head_body,
    grid=(GRID,),
    in_specs=[
        pl.BlockSpec((BLK, HIDDEN), lambda i: (i, 0)),
        pl.BlockSpec((1, BLK, HIDDEN), _agg_map),
        pl.BlockSpec((BLK, 1), lambda i: (i, 0)),
        pl.BlockSpec((1, HIDDEN), lambda i: (0, 0)),
        pl.BlockSpec((HIDDEN, N_CLASSES), lambda i: (0, 0)),
        pl.BlockSpec((1, N_CLASSES), lambda i: (0, 0)),
    ],
    out_specs=pl.BlockSpec((BLK, N_CLASSES), lambda i: (i, 0)),
    out_shape=jax.ShapeDtypeStruct((N_NODES, N_CLASSES), jnp.float32),
)


def kernel(x, edge_index, W1, b1, Wg, bg, W2, b2):
    src = edge_index[0].astype(jnp.int32)
    dst = edge_index[1].astype(jnp.int32)
    npad = E_PAD - N_EDGES
    pad_i = jnp.arange(npad, dtype=jnp.int32)
    src_p = jnp.concatenate([src, pad_i % N_NODES]).reshape(E_ROWS, LANE)
    dst_p = jnp.concatenate([dst, N_NODES + (pad_i & JUNK_MASK)]).reshape(E_ROWS, LANE)

    deg_parts = _deg_kernel(dst_p)
    xw = _xw_call(x, W1.T, b1.reshape(1, HIDDEN), Wg.T)
    deg = jnp.concatenate(
        [deg_parts[0, :NODES_PER_CORE], deg_parts[1, :NODES_PER_CORE]]
    ).reshape(N_NODES, 1)
    y = _scale_call(xw, deg)
    agg_parts = _agg_kernel(y, src_p, dst_p)
    return _head_call(y, agg_parts, deg, bg.reshape(1, HIDDEN),
                      W2.T, b2.reshape(1, N_CLASSES))
